# baseline (speedup 1.0000x reference)
import functools

import jax
import jax.numpy as jnp
from jax import lax
from jax.experimental import pallas as pl
from jax.experimental.pallas import tpu as pltpu
from jax.experimental.pallas import tpu_sc as plsc

NC = 2
NS = 16
NW = NC * NS
L = 16


def _extract(vec_ref, j):
    grp = (j // L) * L
    vec = vec_ref[pl.ds(grp, L)]
    onehot = lax.iota(jnp.int32, L) == (j - grp)
    return jnp.sum(jnp.where(onehot, vec, 0))


@functools.partial(jax.jit, static_argnums=(4, 5, 6))
def _sc_masked_add(x2, a2, gidx, starts, N, D, C):
    RW = N // NW
    nchunk = RW // C

    def body(x_hbm, a_hbm, gidx_hbm, starts_hbm, out_hbm,
             buf, abuf, idxv, stv, sem, gsem):
        cid = lax.axis_index("c")
        sid = lax.axis_index("s")
        w = sid * NC + cid
        base = w * RW
        pltpu.sync_copy(gidx_hbm.at[w], idxv)
        pltpu.sync_copy(starts_hbm.at[w], stv)

        def chunk_body(c, s):
            rowbase = base + c * C
            pltpu.async_copy(x_hbm.at[pl.ds(rowbase, C)], buf, sem).wait()
            e = _extract(stv, c + 1)

            def row_body(j, carry):
                g = _extract(idxv, j)
                p = g - rowbase
                pltpu.async_copy(a_hbm.at[pl.ds(g, 1)], abuf, gsem).wait()

                def add_body(d, c2):
                    sl = pl.ds(d * L, L)
                    plsc.addupdate(buf.at[p, sl], abuf[0, sl])
                    return c2

                lax.fori_loop(0, D // L, add_body, 0, unroll=8)
                return carry

            lax.fori_loop(s, e, row_body, 0)
            pltpu.async_copy(buf, out_hbm.at[pl.ds(rowbase, C)], sem).wait()
            return e

        lax.fori_loop(0, nchunk, chunk_body, 0)

    fn = pl.kernel(
        body,
        out_type=jax.ShapeDtypeStruct((N, D), jnp.float32),
        mesh=plsc.VectorSubcoreMesh(
            core_axis_name="c", subcore_axis_name="s",
            num_cores=NC, num_subcores=NS),
        scratch_types=[
            pltpu.VMEM((C, D), jnp.float32),
            pltpu.VMEM((1, D), jnp.float32),
            pltpu.VMEM((RW,), jnp.int32),
            pltpu.VMEM((64,), jnp.int32),
            pltpu.SemaphoreType.DMA,
            pltpu.SemaphoreType.DMA,
        ],
        compiler_params=pltpu.CompilerParams(needs_layout_passes=False),
    )
    return fn(x2, a2, gidx, starts)


def kernel(x, attack, attack_mask):
    B, S, D = x.shape
    N = B * S
    C = 8
    RW = N // NW
    x2 = x.reshape(N, D)
    a2 = attack.astype(x.dtype).reshape(N, D)
    m2 = attack_mask[:, :S].reshape(NW, RW)
    loc = jnp.argsort(~m2, axis=1, stable=True).astype(jnp.int32)
    gidx = loc + (jnp.arange(NW, dtype=jnp.int32) * RW)[:, None]
    ccnt = m2.reshape(NW, RW // C, C).sum(-1).astype(jnp.int32)
    starts = jnp.concatenate(
        [jnp.zeros((NW, 1), jnp.int32), jnp.cumsum(ccnt, axis=1)], axis=1)
    starts = jnp.pad(starts, ((0, 0), (0, 64 - starts.shape[1])))
    out2 = _sc_masked_add(x2, a2, gidx, starts, N, D, C)
    return out2.reshape(B, S, D)
